# baseline (speedup 1.0000x reference)
import functools

import jax
import jax.numpy as jnp
from jax import lax
from jax.experimental import pallas as pl
from jax.experimental.pallas import tpu as pltpu
from jax.experimental.pallas import tpu_sc as plsc

_INFO = plsc.get_sparse_core_info()
_NC, _NS = _INFO.num_cores, _INFO.num_subcores
_NW = _NC * _NS
_L = 16


@functools.lru_cache(maxsize=None)
def _build(S, B, V, D):
    BW = B // _NW
    TR = D // 8
    H = D // 2
    mesh = plsc.VectorSubcoreMesh(core_axis_name="c", subcore_axis_name="s")

    @functools.partial(
        pl.kernel,
        mesh=mesh,
        out_type=jax.ShapeDtypeStruct((S, TR, _NW, 8 * BW), jnp.float32),
        scratch_types=[
            [pltpu.VMEM((BW,), jnp.int32) for _ in range(2)],
            [pltpu.VMEM((BW,), jnp.int32) for _ in range(2)],
            [pltpu.VMEM((BW, H), jnp.float32) for _ in range(2)],
            [pltpu.VMEM((BW, H), jnp.float32) for _ in range(2)],
            [pltpu.VMEM((TR, 8 * BW), jnp.float32) for _ in range(2)],
            [pltpu.SemaphoreType.DMA for _ in range(2)],
            [pltpu.SemaphoreType.DMA for _ in range(2)],
        ],
        compiler_params=pltpu.CompilerParams(
            use_tc_tiling_on_sc=False, needs_layout_passes=False
        ),
    )
    def gk(tab_hbm, idx_hbm, out_hbm, ji, jj, ga, gb, slab, gsem, osem):
        wid = lax.axis_index("s") * _NC + lax.axis_index("c")
        col0 = wid * BW

        def prep_idx(s, q):
            pltpu.sync_copy(idx_hbm.at[pl.ds(s * B + col0, BW)], ji[q])

            def blk(k, carry):
                v = ji[q][pl.ds(k * _L, _L)]
                jj[q][pl.ds(k * _L, _L)] = (v << 1) + 1
                ji[q][pl.ds(k * _L, _L)] = v << 1
                return carry

            lax.fori_loop(0, BW // _L, blk, 0, unroll=4)

        def g_copies(q):
            return (
                pltpu.make_async_copy(tab_hbm.at[ji[q]], ga[q], gsem[q]),
                pltpu.make_async_copy(tab_hbm.at[jj[q]], gb[q], gsem[q]),
            )

        def wb_copies(s, q):
            return tuple(
                pltpu.make_async_copy(
                    slab[q].at[tr], out_hbm.at[s, tr, wid], osem[q]
                )
                for tr in range(TR)
            )

        def transpose(q):
            def half(src, d0):
                def body(d, carry):
                    dd = d0 + d
                    tr = dd // 8
                    off = (dd % 8) * BW
                    dv = lax.broadcast(d, (_L,))
                    for kb in range(BW // _L):
                        rows = lax.iota(jnp.int32, _L) + (kb * _L)
                        g = plsc.load_gather(src, [rows, dv])
                        slab[q][tr, pl.ds(off + kb * _L, _L)] = g
                    return carry

                lax.fori_loop(0, H, body, 0, unroll=False)

            half(ga[q], 0)
            half(gb[q], H)

        prep_idx(0, 0)
        for cp in g_copies(0):
            cp.start()
        prep_idx(1, 1)

        def step_one(s, q, j):
            for cp in g_copies(q):
                cp.wait()

            @pl.when(s + 1 < S)
            def _():
                for cp in g_copies(1 - q):
                    cp.start()

            @pl.when(j > 0)
            def _():
                for cp in wb_copies(s - 2, q):
                    cp.wait()

            transpose(q)
            for cp in wb_copies(s, q):
                cp.start()

            @pl.when(s + 2 < S)
            def _():
                prep_idx(s + 2, q)

        def step(j, carry):
            step_one(2 * j, 0, j)
            step_one(2 * j + 1, 1, j)
            return carry

        lax.fori_loop(0, S // 2, step, 0, unroll=False)
        for cp in wb_copies(S - 2, 0):
            cp.wait()
        for cp in wb_copies(S - 1, 1):
            cp.wait()

    return gk


def kernel(x, peso):
    B, S = x.shape
    V, D = peso.shape
    tab = peso.reshape(2 * V, D // 2)
    idx = x.T.reshape(S * B)
    out5 = _build(S, B, V, D)(tab, idx)
    out6 = out5.reshape(S, D // 8, _NW, 8, B // _NW)
    return out6.transpose(2, 4, 0, 1, 3).reshape(B, S, D)


# speedup vs baseline: 1.5617x; 1.5617x over previous
import functools

import jax
import jax.numpy as jnp
from jax import lax
from jax.experimental import pallas as pl
from jax.experimental.pallas import tpu as pltpu
from jax.experimental.pallas import tpu_sc as plsc

_INFO = plsc.get_sparse_core_info()
_NC, _NS = _INFO.num_cores, _INFO.num_subcores
_NW = _NC * _NS

_CHUNK = 512


@functools.lru_cache(maxsize=None)
def _build(B, V, D):
    assert B % (_NW * 2 * _CHUNK) == 0
    b_per_w = B // _NW
    n_chunks = b_per_w // _CHUNK
    mesh = plsc.VectorSubcoreMesh(core_axis_name="c", subcore_axis_name="s")

    @functools.partial(
        pl.kernel,
        mesh=mesh,
        out_type=jax.ShapeDtypeStruct((B, D), jnp.float32),
        scratch_types=[
            pltpu.VMEM((b_per_w,), jnp.int32),
            pltpu.VMEM((_CHUNK, D), jnp.float32),
            pltpu.VMEM((_CHUNK, D), jnp.float32),
            pltpu.SemaphoreType.DMA,
            pltpu.SemaphoreType.DMA,
            pltpu.SemaphoreType.DMA,
            pltpu.SemaphoreType.DMA,
        ],
        compiler_params=pltpu.CompilerParams(use_tc_tiling_on_sc=False),
    )
    def gather_kernel(table_hbm, idx_hbm, out_hbm, idx_v, r0, r1, g0, g1, o0, o1):
        rows = (r0, r1)
        gsem = (g0, g1)
        osem = (o0, o1)
        wid = lax.axis_index("s") * _NC + lax.axis_index("c")
        base = wid * b_per_w

        pltpu.sync_copy(idx_hbm.at[pl.ds(base, b_per_w)], idx_v)

        def gather_copy(c, b):
            return pltpu.make_async_copy(
                table_hbm.at[idx_v.at[pl.ds(c * _CHUNK, _CHUNK)]],
                rows[b],
                gsem[b],
            )

        def out_copy(c, b):
            return pltpu.make_async_copy(
                rows[b],
                out_hbm.at[pl.ds(base + c * _CHUNK, _CHUNK)],
                osem[b],
            )

        gather_copy(0, 0).start()

        def step(j, carry):
            c0 = 2 * j
            gather_copy(c0, 0).wait()
            out_copy(c0, 0).start()
            @pl.when(j > 0)
            def _():
                out_copy(c0 - 1, 1).wait()

            gather_copy(c0 + 1, 1).start()

            gather_copy(c0 + 1, 1).wait()
            out_copy(c0 + 1, 1).start()
            out_copy(c0, 0).wait()

            @pl.when(j < n_chunks // 2 - 1)
            def _():
                gather_copy(c0 + 2, 0).start()

            return carry

        lax.fori_loop(0, n_chunks // 2, step, 0, unroll=False)
        out_copy(n_chunks - 1, 1).wait()

    return gather_kernel


def kernel(x, peso):
    B0, S = x.shape
    V, D = peso.shape
    flat_idx = x.reshape(B0 * S)
    out = _build(B0 * S, V, D)(peso, flat_idx)
    return out.reshape(B0, S, D)
